# Initial kernel scaffold; baseline (speedup 1.0000x reference)
import functools

import jax
import jax.numpy as jnp
from jax import lax
from jax.experimental import pallas as pl
from jax.experimental.pallas import tpu as pltpu
from jax.experimental.pallas import tpu_sc as plsc

_NB = 1000


def _tc_a(x_ref, w_ref, al_ref, ar_ref, h_ref, el_ref, er_ref, gm_ref):
    heads = w_ref.shape[0]

    @pl.when(pl.program_id(0) == 0)
    def _():
        gm_ref[...] = jnp.full(gm_ref.shape, -jnp.inf, jnp.float32)

    for k in range(heads):
        hk = jnp.dot(x_ref[...], w_ref[k], preferred_element_type=jnp.float32)
        h_ref[k] = hk
        el = (hk * al_ref[k][None, :]).sum(axis=-1)
        er = (hk * ar_ref[k][None, :]).sum(axis=-1)
        el_ref[k] = el
        er_ref[k] = er
        gm_ref[0, k] = jnp.maximum(gm_ref[0, k], jnp.max(el))
        gm_ref[1, k] = jnp.maximum(gm_ref[1, k], jnp.max(er))


def _tc_c(unn_ref, es_ref, b1_ref, w2_ref, al2_ref, ar2_ref,
          h2_ref, elr_ref, gm2_ref):
    heads = unn_ref.shape[0]

    @pl.when(pl.program_id(0) == 0)
    def _():
        gm2_ref[...] = jnp.full(gm2_ref.shape, -jnp.inf, jnp.float32)

    parts = es_ref[...]
    acc = jnp.zeros((unn_ref.shape[1], w2_ref.shape[2]), jnp.float32)
    for k in range(heads):
        c, k4 = k // 4, k % 4
        esum_k = (parts[c * 16 + k4] + parts[c * 16 + 4 + k4]
                  + parts[c * 16 + 8 + k4] + parts[c * 16 + 12 + k4])
        inv_k = 1.0 / (esum_k + 1e-9)
        o1 = unn_ref[k] * inv_k[:, None] + b1_ref[k][None, :]
        z = jnp.where(o1 > 0, o1, jnp.expm1(o1))
        acc = acc + jnp.dot(z, w2_ref[k], preferred_element_type=jnp.float32)
    h2_ref[...] = acc
    el2 = (acc * al2_ref[0][None, :]).sum(axis=-1)
    er2 = (acc * ar2_ref[0][None, :]).sum(axis=-1)
    elr_ref[0] = el2
    elr_ref[1] = er2
    gm2_ref[0] = jnp.maximum(gm2_ref[0], jnp.max(el2))
    gm2_ref[1] = jnp.maximum(gm2_ref[1], jnp.max(er2))


def _tc_e(unn2_ref, es2_ref, b2_ref, out_ref):
    esum = jnp.sum(es2_ref[...], axis=0)
    inv = 1.0 / (esum + 1e-9)
    u = unn2_ref[0] + unn2_ref[1]
    out_ref[...] = u * inv[:, None] + b2_ref[0][None, :]


def _leaky_shift(a, b):
    g = a + b
    return jnp.where(g >= 0.0, g, 0.2 * g)


def _zero16():
    return jnp.zeros((16,), jnp.float32)


def _sc_b(n, e, hid, elT, erT, gm, src, dst, h1f, zin,
          exT, esumP, unn,
          el_v, er_v, esum_v, gm_v, s1_v, d1_v, x1_v,
          s2_v, d2_v, x2_v, i2_v, rows_v, zbuf_v, acc_sh, sem):
    c = lax.axis_index("c")
    s = lax.axis_index("s")
    g = c * 16 + s
    C1 = 2000
    ept1 = e // 4
    nc1 = ept1 // C1

    k_loc = lax.rem(s, 4)
    q = lax.div(s, 4)
    head1 = c * 4 + k_loc
    pltpu.sync_copy(gm, gm_v)
    pltpu.sync_copy(zin, zbuf_v)
    pltpu.sync_copy(elT.at[pl.ds(head1 * n, n)], el_v)
    pltpu.sync_copy(erT.at[pl.ds(head1 * n, n)], er_v)

    def _z(i, _):
        esum_v[pl.ds(i * 16, 16)] = _zero16()
        return 0
    lax.fori_loop(0, n // 16, _z, 0)

    gmax1 = _leaky_shift(gm_v[0, head1, 0], gm_v[1, head1, 0])

    def _chunk1(i, _):
        b = q * ept1 + i * C1
        pltpu.sync_copy(src.at[pl.ds(b, C1)], s1_v)
        pltpu.sync_copy(dst.at[pl.ds(b, C1)], d1_v)

        def _inner(j, _):
            s16 = s1_v[pl.ds(j * 16, 16)]
            d16 = d1_v[pl.ds(j * 16, 16)]
            ev = plsc.load_gather(el_v, [s16]) + plsc.load_gather(er_v, [d16])
            ev = jnp.where(ev >= 0.0, ev, 0.2 * ev)
            ex = jnp.exp(ev - gmax1)
            x1_v[pl.ds(j * 16, 16)] = ex
            plsc.addupdate_scatter(esum_v, [d16], ex)
            return 0
        lax.fori_loop(0, C1 // 16, _inner, 0)
        pltpu.sync_copy(x1_v, exT.at[pl.ds(head1 * e + b, C1)])
        return 0
    lax.fori_loop(0, nc1, _chunk1, 0)
    pltpu.sync_copy(esum_v, esumP.at[pl.ds(g * n, n)])

    plsc.subcore_barrier()

    CH = 128
    nch = e // CH
    nmine = lax.div(nch, 16) + jnp.where(s < lax.rem(nch, 16), 1, 0)
    npt = n // 16

    for k4 in range(4):
        head = c * 4 + k4
        pltpu.sync_copy(zbuf_v, acc_sh.at[pl.ds(s * npt, npt)])
        plsc.subcore_barrier()

        def _chunk2(i, _):
            b = (s + i * 16) * CH
            pltpu.sync_copy(src.at[pl.ds(b, CH)], s2_v)
            pltpu.sync_copy(dst.at[pl.ds(b, CH)], d2_v)
            pltpu.sync_copy(exT.at[pl.ds(head * e + b, CH)], x2_v)
            off = head * n

            def _addi(j, _):
                i2_v[pl.ds(j * 16, 16)] = s2_v[pl.ds(j * 16, 16)] + off
                return 0
            lax.fori_loop(0, CH // 16, _addi, 0)
            pltpu.async_copy(h1f.at[i2_v], rows_v, sem).wait()

            def _mrow(r, _):
                sc = x2_v[r]
                for jj in range(hid // 16):
                    sl = pl.ds(jj * 16, 16)
                    rows_v[r, sl] = rows_v[r, sl] * sc
                return 0
            lax.fori_loop(0, CH, _mrow, 0)
            pltpu.sync_copy(rows_v, acc_sh.at[d2_v], add=True)
            return 0
        lax.fori_loop(0, nmine, _chunk2, 0)
        plsc.subcore_barrier()
        pltpu.sync_copy(acc_sh.at[pl.ds(s * npt, npt)],
                        unn.at[pl.ds(head * n + s * npt, npt)])


def _sc_d(n, e, hid, elr2, gm2, src, dst, h2, zin,
          exT2, esumP2, unn2,
          el_v, er_v, esum_v, gm_v, s1_v, d1_v, x1_v,
          s2_v, d2_v, x2_v, rows_v, zbuf_v, acc_sh, sem):
    c = lax.axis_index("c")
    s = lax.axis_index("s")
    g = c * 16 + s
    C1 = 2000
    ept1 = e // 32
    nc1 = ept1 // C1

    pltpu.sync_copy(gm2, gm_v)
    pltpu.sync_copy(zin, zbuf_v)
    pltpu.sync_copy(elr2.at[pl.ds(0, n)], el_v)
    pltpu.sync_copy(elr2.at[pl.ds(n, n)], er_v)

    def _z(i, _):
        esum_v[pl.ds(i * 16, 16)] = _zero16()
        return 0
    lax.fori_loop(0, n // 16, _z, 0)

    gmax2 = _leaky_shift(gm_v[0, 0], gm_v[1, 0])

    def _chunk1(i, _):
        b = g * ept1 + i * C1
        pltpu.sync_copy(src.at[pl.ds(b, C1)], s1_v)
        pltpu.sync_copy(dst.at[pl.ds(b, C1)], d1_v)

        def _inner(j, _):
            s16 = s1_v[pl.ds(j * 16, 16)]
            d16 = d1_v[pl.ds(j * 16, 16)]
            ev = plsc.load_gather(el_v, [s16]) + plsc.load_gather(er_v, [d16])
            ev = jnp.where(ev >= 0.0, ev, 0.2 * ev)
            ex = jnp.exp(ev - gmax2)
            x1_v[pl.ds(j * 16, 16)] = ex
            plsc.addupdate_scatter(esum_v, [d16], ex)
            return 0
        lax.fori_loop(0, C1 // 16, _inner, 0)
        pltpu.sync_copy(x1_v, exT2.at[pl.ds(b, C1)])
        return 0
    lax.fori_loop(0, nc1, _chunk1, 0)
    pltpu.sync_copy(esum_v, esumP2.at[pl.ds(g * n, n)])

    plsc.subcore_barrier()

    CH = 128
    nch = (e // 2) // CH
    nmine = lax.div(nch, 16) + jnp.where(s < lax.rem(nch, 16), 1, 0)
    npt = n // 16

    pltpu.sync_copy(zbuf_v, acc_sh.at[pl.ds(s * npt, npt)])
    plsc.subcore_barrier()

    def _chunk2(i, _):
        b = (c * nch + s + i * 16) * CH
        pltpu.sync_copy(src.at[pl.ds(b, CH)], s2_v)
        pltpu.sync_copy(dst.at[pl.ds(b, CH)], d2_v)
        pltpu.sync_copy(exT2.at[pl.ds(b, CH)], x2_v)
        pltpu.async_copy(h2.at[s2_v], rows_v, sem).wait()

        def _mrow(r, _):
            sc = x2_v[r]
            for jj in range(hid // 16):
                sl = pl.ds(jj * 16, 16)
                rows_v[r, sl] = rows_v[r, sl] * sc
            return 0
        lax.fori_loop(0, CH, _mrow, 0)
        pltpu.sync_copy(rows_v, acc_sh.at[d2_v], add=True)
        return 0
    lax.fori_loop(0, nmine, _chunk2, 0)
    plsc.subcore_barrier()
    pltpu.sync_copy(acc_sh.at[pl.ds(s * npt, npt)],
                    unn2.at[pl.ds(c * n + s * npt, npt)])


def kernel(inputs, edge_index, W1, attn_l1, attn_r1, b1,
           W2, attn_l2, attn_r2, b2):
    n, in_size = inputs.shape
    e = edge_index.shape[1]
    h1h, hid = attn_l1.shape
    out_dim = attn_l2.shape[1]
    src = edge_index[0]
    dst = edge_index[1]
    w1r = W1.reshape(in_size, h1h, hid).transpose(1, 0, 2)
    w2r = W2.reshape(h1h, hid, out_dim)
    b1r = b1.reshape(h1h, hid)
    b2r = b2.reshape(1, out_dim)
    grid = n // _NB
    f32 = jnp.float32

    h1, elT, erT, gm1 = pl.pallas_call(
        _tc_a,
        grid=(grid,),
        in_specs=[
            pl.BlockSpec((_NB, in_size), lambda i: (i, 0)),
            pl.BlockSpec((h1h, in_size, hid), lambda i: (0, 0, 0)),
            pl.BlockSpec((h1h, hid), lambda i: (0, 0)),
            pl.BlockSpec((h1h, hid), lambda i: (0, 0)),
        ],
        out_specs=[
            pl.BlockSpec((h1h, _NB, hid), lambda i: (0, i, 0)),
            pl.BlockSpec((h1h, _NB), lambda i: (0, i)),
            pl.BlockSpec((h1h, _NB), lambda i: (0, i)),
            pl.BlockSpec((2, h1h, 128), lambda i: (0, 0, 0)),
        ],
        out_shape=[
            jax.ShapeDtypeStruct((h1h, n, hid), f32),
            jax.ShapeDtypeStruct((h1h, n), f32),
            jax.ShapeDtypeStruct((h1h, n), f32),
            jax.ShapeDtypeStruct((2, h1h, 128), f32),
        ],
    )(inputs, w1r, attn_l1, attn_r1)

    h1f = h1.reshape(h1h * n, hid)
    zin = jnp.zeros((n // 16, hid), f32)

    mesh = plsc.VectorSubcoreMesh(core_axis_name="c", subcore_axis_name="s")
    scb = pl.kernel(
        functools.partial(_sc_b, n, e, hid),
        out_type=[
            jax.ShapeDtypeStruct((h1h * e,), f32),
            jax.ShapeDtypeStruct((32 * n,), f32),
            jax.ShapeDtypeStruct((h1h * n, hid), f32),
        ],
        mesh=mesh,
        scratch_types=[
            pltpu.VMEM((n,), f32),
            pltpu.VMEM((n,), f32),
            pltpu.VMEM((n,), f32),
            pltpu.VMEM((2, h1h, 128), f32),
            pltpu.VMEM((2000,), jnp.int32),
            pltpu.VMEM((2000,), jnp.int32),
            pltpu.VMEM((2000,), f32),
            pltpu.VMEM((128,), jnp.int32),
            pltpu.VMEM((128,), jnp.int32),
            pltpu.VMEM((128,), f32),
            pltpu.VMEM((128,), jnp.int32),
            pltpu.VMEM((128, hid), f32),
            pltpu.VMEM((n // 16, hid), f32),
            pltpu.VMEM_SHARED((n, hid), f32),
            pltpu.SemaphoreType.DMA,
        ],
    )
    exT, esumP, unn = scb(elT.reshape(-1), erT.reshape(-1), gm1, src, dst,
                          h1f, zin)

    h2, elr2, gm2 = pl.pallas_call(
        _tc_c,
        grid=(grid,),
        in_specs=[
            pl.BlockSpec((h1h, _NB, hid), lambda i: (0, i, 0)),
            pl.BlockSpec((32, _NB), lambda i: (0, i)),
            pl.BlockSpec((h1h, hid), lambda i: (0, 0)),
            pl.BlockSpec((h1h, hid, out_dim), lambda i: (0, 0, 0)),
            pl.BlockSpec((1, out_dim), lambda i: (0, 0)),
            pl.BlockSpec((1, out_dim), lambda i: (0, 0)),
        ],
        out_specs=[
            pl.BlockSpec((_NB, out_dim), lambda i: (i, 0)),
            pl.BlockSpec((2, _NB), lambda i: (0, i)),
            pl.BlockSpec((2, 128), lambda i: (0, 0)),
        ],
        out_shape=[
            jax.ShapeDtypeStruct((n, out_dim), f32),
            jax.ShapeDtypeStruct((2, n), f32),
            jax.ShapeDtypeStruct((2, 128), f32),
        ],
    )(unn.reshape(h1h, n, hid), esumP.reshape(32, n), b1r, w2r,
      attn_l2, attn_r2)

    scd = pl.kernel(
        functools.partial(_sc_d, n, e, out_dim),
        out_type=[
            jax.ShapeDtypeStruct((e,), f32),
            jax.ShapeDtypeStruct((32 * n,), f32),
            jax.ShapeDtypeStruct((2 * n, out_dim), f32),
        ],
        mesh=mesh,
        scratch_types=[
            pltpu.VMEM((n,), f32),
            pltpu.VMEM((n,), f32),
            pltpu.VMEM((n,), f32),
            pltpu.VMEM((2, 128), f32),
            pltpu.VMEM((2000,), jnp.int32),
            pltpu.VMEM((2000,), jnp.int32),
            pltpu.VMEM((2000,), f32),
            pltpu.VMEM((128,), jnp.int32),
            pltpu.VMEM((128,), jnp.int32),
            pltpu.VMEM((128,), f32),
            pltpu.VMEM((128, out_dim), f32),
            pltpu.VMEM((n // 16, out_dim), f32),
            pltpu.VMEM_SHARED((n, out_dim), f32),
            pltpu.SemaphoreType.DMA,
        ],
    )
    exT2, esumP2, unn2 = scd(elr2.reshape(-1), gm2, src, dst, h2, zin)

    out = pl.pallas_call(
        _tc_e,
        grid=(grid,),
        in_specs=[
            pl.BlockSpec((2, _NB, out_dim), lambda i: (0, i, 0)),
            pl.BlockSpec((32, _NB), lambda i: (0, i)),
            pl.BlockSpec((1, out_dim), lambda i: (0, 0)),
        ],
        out_specs=pl.BlockSpec((_NB, out_dim), lambda i: (i, 0)),
        out_shape=jax.ShapeDtypeStruct((n, out_dim), f32),
    )(unn2.reshape(2, n, out_dim), esumP2.reshape(32, n), b2r)

    return out


# speedup vs baseline: 15.7739x; 15.7739x over previous
import functools

import jax
import jax.numpy as jnp
from jax import lax
from jax.experimental import pallas as pl
from jax.experimental.pallas import tpu as pltpu
from jax.experimental.pallas import tpu_sc as plsc

_NB = 1000


def _tc_a(x_ref, w_ref, al_ref, ar_ref, h_ref, el_ref, er_ref, gm_ref):
    heads = w_ref.shape[0]

    @pl.when(pl.program_id(0) == 0)
    def _():
        gm_ref[...] = jnp.full(gm_ref.shape, -jnp.inf, jnp.float32)

    for k in range(heads):
        hk = jnp.dot(x_ref[...], w_ref[k], preferred_element_type=jnp.float32)
        cb = (k % 2) * 64
        h_ref[k // 2, :, cb:cb + 64] = hk
        el = (hk * al_ref[k][None, :]).sum(axis=-1)
        er = (hk * ar_ref[k][None, :]).sum(axis=-1)
        el_ref[k, 0, 0] = el
        er_ref[k, 0, 0] = er
        gm_ref[0, k] = jnp.maximum(gm_ref[0, k], jnp.max(el))
        gm_ref[1, k] = jnp.maximum(gm_ref[1, k], jnp.max(er))


def _tc_c(unn_ref, es_ref, b1_ref, w2_ref, al2_ref, ar2_ref,
          h2_ref, elr_ref, gm2_ref):
    heads = 2 * unn_ref.shape[0]

    @pl.when(pl.program_id(0) == 0)
    def _():
        gm2_ref[...] = jnp.full(gm2_ref.shape, -jnp.inf, jnp.float32)

    parts = es_ref[...]
    acc = jnp.zeros((unn_ref.shape[1], w2_ref.shape[2]), jnp.float32)
    for k in range(heads):
        c, k4 = k // 4, k % 4
        esum_k = (parts[c * 16 + k4, 0, 0] + parts[c * 16 + 4 + k4, 0, 0]
                  + parts[c * 16 + 8 + k4, 0, 0] + parts[c * 16 + 12 + k4, 0, 0])
        inv_k = 1.0 / (esum_k + 1e-9)
        cb = (k % 2) * 64
        o1 = (unn_ref[k // 2, :, cb:cb + 64] * inv_k[:, None]
              + b1_ref[k][None, :])
        z = jnp.where(o1 > 0, o1, jnp.exp(o1) - 1.0)
        acc = acc + jnp.dot(z, w2_ref[k], preferred_element_type=jnp.float32)
    h2_ref[:, 0:64] = acc
    h2_ref[:, 64:128] = jnp.zeros_like(acc)
    el2 = (acc * al2_ref[0][None, :]).sum(axis=-1)
    er2 = (acc * ar2_ref[0][None, :]).sum(axis=-1)
    elr_ref[0, 0, 0] = el2
    elr_ref[1, 0, 0] = er2
    gm2_ref[0] = jnp.maximum(gm2_ref[0], jnp.max(el2))
    gm2_ref[1] = jnp.maximum(gm2_ref[1], jnp.max(er2))


def _tc_e(unn2_ref, es2_ref, b2_ref, out_ref):
    esum = jnp.sum(es2_ref[...], axis=(0, 1, 2))
    inv = 1.0 / (esum + 1e-9)
    u = unn2_ref[0, :, 0:64] + unn2_ref[1, :, 0:64]
    out_ref[...] = u * inv[:, None] + b2_ref[0][None, :]


def _leaky_shift(a, b):
    g = a + b
    return jnp.where(g >= 0.0, g, 0.2 * g)


def _zero16():
    return jnp.zeros((16,), jnp.float32)


def _acc_zero(s, zin, acc_sh, sm, lg):
    @pl.when(s < 15)
    def _():
        pltpu.sync_copy(zin.at[pl.ds(0, sm)], acc_sh.at[pl.ds(s * sm, sm)])

    @pl.when(s == 15)
    def _():
        pltpu.sync_copy(zin, acc_sh.at[pl.ds(15 * sm, lg)])


def _acc_out(s, acc_sh, dst_ref, base, sm, lg):
    @pl.when(s < 15)
    def _():
        pltpu.sync_copy(acc_sh.at[pl.ds(s * sm, sm)],
                        dst_ref.at[pl.ds(base + s * sm, sm)])

    @pl.when(s == 15)
    def _():
        pltpu.sync_copy(acc_sh.at[pl.ds(15 * sm, lg)],
                        dst_ref.at[pl.ds(base + 15 * sm, lg)])


def _phase1(n, C1, ept1, base0, head_e_off, gmax, el_v, er_v, esum_v,
            s1_v, d1_v, x1_v, src, dst, exT):
    def _z(i, _):
        esum_v[pl.ds(i * 16, 16)] = _zero16()
        return 0
    lax.fori_loop(0, n // 16, _z, 0)

    def _chunk1(i, _):
        b = base0 + i * C1
        pltpu.sync_copy(src.at[pl.ds(b, C1)], s1_v)
        pltpu.sync_copy(dst.at[pl.ds(b, C1)], d1_v)

        def _inner(j, _):
            s16 = s1_v[pl.ds(j * 16, 16)]
            d16 = d1_v[pl.ds(j * 16, 16)]
            ev = plsc.load_gather(el_v, [s16]) + plsc.load_gather(er_v, [d16])
            ev = jnp.where(ev >= 0.0, ev, 0.2 * ev)
            ex = jnp.exp(ev - gmax)
            x1_v[pl.ds(j * 16, 16)] = ex
            plsc.addupdate_scatter(esum_v, [d16], ex)
            return 0
        lax.fori_loop(0, C1 // 16, _inner, 0)
        pltpu.sync_copy(x1_v, exT.at[pl.ds(head_e_off + b, C1)])
        return 0
    lax.fori_loop(0, ept1 // C1, _chunk1, 0)


def _sc_b(n, e, elT, erT, gm, src, dst, h1p, zin,
          exT, esumP, unn,
          el_v, er_v, esum_v, gm_v, s1_v, d1_v, x1_v,
          s2_v, d2_v, xa_v, xb_v, i2_v, rows_v, acc_sh, sem):
    c = lax.axis_index("c")
    s = lax.axis_index("s")
    g = c * 16 + s
    C1 = 2000
    ept1 = e // 4

    k_loc = lax.rem(s, 4)
    q = lax.div(s, 4)
    head1 = c * 4 + k_loc
    pltpu.sync_copy(gm.at[pl.ds(head1 * 128, 16)], gm_v.at[pl.ds(0, 16)])
    pltpu.sync_copy(gm.at[pl.ds(1024 + head1 * 128, 16)],
                    gm_v.at[pl.ds(16, 16)])
    pltpu.sync_copy(elT.at[pl.ds(head1 * n, n)], el_v)
    pltpu.sync_copy(erT.at[pl.ds(head1 * n, n)], er_v)
    gmax1 = _leaky_shift(gm_v[pl.ds(0, 16)], gm_v[pl.ds(16, 16)])
    _phase1(n, C1, ept1, q * ept1, head1 * e, gmax1, el_v, er_v, esum_v,
            s1_v, d1_v, x1_v, src, dst, exT)
    pltpu.sync_copy(esum_v, esumP.at[pl.ds(g * n, n)])

    plsc.subcore_barrier()

    CH = 64
    nch = e // CH
    nmine = lax.div(nch, 16) + jnp.where(s < lax.rem(nch, 16), 1, 0)
    sm = (n // 16) & ~7
    lg = n - 15 * sm

    for p2 in range(2):
        pair = c * 2 + p2
        _acc_zero(s, zin, acc_sh, sm, lg)
        plsc.subcore_barrier()

        def _chunk2(i, _):
            b = (s + i * 16) * CH
            pltpu.sync_copy(src.at[pl.ds(b, CH)], s2_v)
            pltpu.sync_copy(dst.at[pl.ds(b, CH)], d2_v)
            pltpu.sync_copy(exT.at[pl.ds(2 * pair * e + b, CH)], xa_v)
            pltpu.sync_copy(exT.at[pl.ds((2 * pair + 1) * e + b, CH)], xb_v)
            off = pair * n

            def _addi(j, _):
                i2_v[pl.ds(j * 16, 16)] = s2_v[pl.ds(j * 16, 16)] + off
                return 0
            lax.fori_loop(0, CH // 16, _addi, 0)
            pltpu.async_copy(h1p.at[i2_v], rows_v, sem).wait()

            def _mrow(r, _):
                ridx = jnp.zeros((16,), jnp.int32) + r
                sca = plsc.load_gather(xa_v, [ridx])
                scb = plsc.load_gather(xb_v, [ridx])
                for jj in range(4):
                    sl = pl.ds(jj * 16, 16)
                    rows_v[r, sl] = rows_v[r, sl] * sca
                for jj in range(4, 8):
                    sl = pl.ds(jj * 16, 16)
                    rows_v[r, sl] = rows_v[r, sl] * scb
                return 0
            lax.fori_loop(0, CH, _mrow, 0)
            pltpu.sync_copy(rows_v, acc_sh.at[d2_v], add=True)
            return 0
        lax.fori_loop(0, nmine, _chunk2, 0)
        plsc.subcore_barrier()
        _acc_out(s, acc_sh, unn, pair * n, sm, lg)
        plsc.subcore_barrier()


def _sc_d(n, e, elr2, gm2, src, dst, h2p, zin,
          exT2, esumP2, unn2,
          el_v, er_v, esum_v, gm_v, s1_v, d1_v, x1_v,
          s2_v, d2_v, xa_v, rows_v, acc_sh, sem):
    c = lax.axis_index("c")
    s = lax.axis_index("s")
    g = c * 16 + s
    C1 = 2000
    ept1 = e // 32

    pltpu.sync_copy(gm2.at[pl.ds(0, 16)], gm_v.at[pl.ds(0, 16)])
    pltpu.sync_copy(gm2.at[pl.ds(128, 16)], gm_v.at[pl.ds(16, 16)])
    pltpu.sync_copy(elr2.at[pl.ds(0, n)], el_v)
    pltpu.sync_copy(elr2.at[pl.ds(n, n)], er_v)
    gmax2 = _leaky_shift(gm_v[pl.ds(0, 16)], gm_v[pl.ds(16, 16)])
    _phase1(n, C1, ept1, g * ept1, 0, gmax2, el_v, er_v, esum_v,
            s1_v, d1_v, x1_v, src, dst, exT2)
    pltpu.sync_copy(esum_v, esumP2.at[pl.ds(g * n, n)])

    plsc.subcore_barrier()

    CH = 64
    nch = (e // 2) // CH
    nmine = lax.div(nch, 16) + jnp.where(s < lax.rem(nch, 16), 1, 0)
    sm = (n // 16) & ~7
    lg = n - 15 * sm

    _acc_zero(s, zin, acc_sh, sm, lg)
    plsc.subcore_barrier()

    def _chunk2(i, _):
        b = (c * nch + s + i * 16) * CH
        pltpu.sync_copy(src.at[pl.ds(b, CH)], s2_v)
        pltpu.sync_copy(dst.at[pl.ds(b, CH)], d2_v)
        pltpu.sync_copy(exT2.at[pl.ds(b, CH)], xa_v)
        pltpu.async_copy(h2p.at[s2_v], rows_v, sem).wait()

        def _mrow(r, _):
            ridx = jnp.zeros((16,), jnp.int32) + r
            sca = plsc.load_gather(xa_v, [ridx])
            for jj in range(8):
                sl = pl.ds(jj * 16, 16)
                rows_v[r, sl] = rows_v[r, sl] * sca
            return 0
        lax.fori_loop(0, CH, _mrow, 0)
        pltpu.sync_copy(rows_v, acc_sh.at[d2_v], add=True)
        return 0
    lax.fori_loop(0, nmine, _chunk2, 0)
    plsc.subcore_barrier()
    _acc_out(s, acc_sh, unn2, c * n, sm, lg)


def kernel(inputs, edge_index, W1, attn_l1, attn_r1, b1,
           W2, attn_l2, attn_r2, b2):
    n, in_size = inputs.shape
    e = edge_index.shape[1]
    h1h, hid = attn_l1.shape
    out_dim = attn_l2.shape[1]
    src = edge_index[0]
    dst = edge_index[1]
    w1r = W1.reshape(in_size, h1h, hid).transpose(1, 0, 2)
    w2r = W2.reshape(h1h, hid, out_dim)
    b1r = b1.reshape(h1h, hid)
    b2r = b2.reshape(1, out_dim)
    grid = n // _NB
    f32 = jnp.float32
    i32 = jnp.int32

    h1p, elT, erT, gm1 = pl.pallas_call(
        _tc_a,
        grid=(grid,),
        in_specs=[
            pl.BlockSpec((_NB, in_size), lambda i: (i, 0)),
            pl.BlockSpec((h1h, in_size, hid), lambda i: (0, 0, 0)),
            pl.BlockSpec((h1h, hid), lambda i: (0, 0)),
            pl.BlockSpec((h1h, hid), lambda i: (0, 0)),
        ],
        out_specs=[
            pl.BlockSpec((h1h // 2, _NB, 128), lambda i: (0, i, 0)),
            pl.BlockSpec((h1h, 1, 1, _NB), lambda i: (0, i, 0, 0)),
            pl.BlockSpec((h1h, 1, 1, _NB), lambda i: (0, i, 0, 0)),
            pl.BlockSpec((2, h1h, 128), lambda i: (0, 0, 0)),
        ],
        out_shape=[
            jax.ShapeDtypeStruct((h1h // 2, n, 128), f32),
            jax.ShapeDtypeStruct((h1h, grid, 1, _NB), f32),
            jax.ShapeDtypeStruct((h1h, grid, 1, _NB), f32),
            jax.ShapeDtypeStruct((2, h1h, 128), f32),
        ],
    )(inputs, w1r, attn_l1, attn_r1)

    sm0 = (n // 16) & ~7
    lgz = n - 15 * sm0
    zin = jnp.zeros((lgz, 128), f32)

    mesh = plsc.VectorSubcoreMesh(core_axis_name="c", subcore_axis_name="s")
    scb = pl.kernel(
        functools.partial(_sc_b, n, e),
        out_type=[
            jax.ShapeDtypeStruct((h1h * e,), f32),
            jax.ShapeDtypeStruct((32 * n,), f32),
            jax.ShapeDtypeStruct((h1h // 2 * n, 128), f32),
        ],
        mesh=mesh,
        compiler_params=pltpu.CompilerParams(needs_layout_passes=False),
        scratch_types=[
            pltpu.VMEM((n,), f32),
            pltpu.VMEM((n,), f32),
            pltpu.VMEM((n,), f32),
            pltpu.VMEM((32,), f32),
            pltpu.VMEM((2000,), i32),
            pltpu.VMEM((2000,), i32),
            pltpu.VMEM((2000,), f32),
            pltpu.VMEM((64,), i32),
            pltpu.VMEM((64,), i32),
            pltpu.VMEM((64,), f32),
            pltpu.VMEM((64,), f32),
            pltpu.VMEM((64,), i32),
            pltpu.VMEM((64, 128), f32),
            pltpu.VMEM_SHARED((n, 128), f32),
            pltpu.SemaphoreType.DMA,
        ],
    )
    exT, esumP, unn = scb(elT.reshape(-1), erT.reshape(-1), gm1.reshape(-1), src, dst,
                          h1p.reshape(h1h // 2 * n, 128), zin)

    h2p, elr2, gm2 = pl.pallas_call(
        _tc_c,
        grid=(grid,),
        in_specs=[
            pl.BlockSpec((h1h // 2, _NB, 128), lambda i: (0, i, 0)),
            pl.BlockSpec((32, 1, 1, _NB), lambda i: (0, i, 0, 0)),
            pl.BlockSpec((h1h, hid), lambda i: (0, 0)),
            pl.BlockSpec((h1h, hid, out_dim), lambda i: (0, 0, 0)),
            pl.BlockSpec((1, out_dim), lambda i: (0, 0)),
            pl.BlockSpec((1, out_dim), lambda i: (0, 0)),
        ],
        out_specs=[
            pl.BlockSpec((_NB, 128), lambda i: (i, 0)),
            pl.BlockSpec((2, 1, 1, _NB), lambda i: (0, i, 0, 0)),
            pl.BlockSpec((2, 128), lambda i: (0, 0)),
        ],
        out_shape=[
            jax.ShapeDtypeStruct((n, 128), f32),
            jax.ShapeDtypeStruct((2, grid, 1, _NB), f32),
            jax.ShapeDtypeStruct((2, 128), f32),
        ],
    )(unn.reshape(h1h // 2, n, 128), esumP.reshape(32, grid, 1, _NB),
      b1r, w2r, attn_l2, attn_r2)

    scd = pl.kernel(
        functools.partial(_sc_d, n, e),
        out_type=[
            jax.ShapeDtypeStruct((e,), f32),
            jax.ShapeDtypeStruct((32 * n,), f32),
            jax.ShapeDtypeStruct((2 * n, 128), f32),
        ],
        mesh=mesh,
        compiler_params=pltpu.CompilerParams(needs_layout_passes=False),
        scratch_types=[
            pltpu.VMEM((n,), f32),
            pltpu.VMEM((n,), f32),
            pltpu.VMEM((n,), f32),
            pltpu.VMEM((32,), f32),
            pltpu.VMEM((2000,), i32),
            pltpu.VMEM((2000,), i32),
            pltpu.VMEM((2000,), f32),
            pltpu.VMEM((64,), i32),
            pltpu.VMEM((64,), i32),
            pltpu.VMEM((64,), f32),
            pltpu.VMEM((64, 128), f32),
            pltpu.VMEM_SHARED((n, 128), f32),
            pltpu.SemaphoreType.DMA,
        ],
    )
    exT2, esumP2, unn2 = scd(elr2.reshape(-1), gm2.reshape(-1), src, dst, h2p, zin)

    out = pl.pallas_call(
        _tc_e,
        grid=(grid,),
        in_specs=[
            pl.BlockSpec((2, _NB, 128), lambda i: (0, i, 0)),
            pl.BlockSpec((32, 1, 1, _NB), lambda i: (0, i, 0, 0)),
            pl.BlockSpec((1, out_dim), lambda i: (0, 0)),
        ],
        out_specs=pl.BlockSpec((_NB, out_dim), lambda i: (i, 0)),
        out_shape=jax.ShapeDtypeStruct((n, out_dim), f32),
    )(unn2.reshape(2, n, 128), esumP2.reshape(32, grid, 1, _NB), b2r)

    return out


# speedup vs baseline: 38.4960x; 2.4405x over previous
import functools

import jax
import jax.numpy as jnp
from jax import lax
from jax.experimental import pallas as pl
from jax.experimental.pallas import tpu as pltpu
from jax.experimental.pallas import tpu_sc as plsc

_NB = 1000
_C1 = 2000
_C2 = 1280
_SUB = 128
_NSUB = _C2 // _SUB


def _tc_a(x_ref, w_ref, al_ref, ar_ref, h_ref, el_ref, er_ref, gm_ref):
    heads = w_ref.shape[0]

    @pl.when(pl.program_id(0) == 0)
    def _():
        gm_ref[...] = jnp.full(gm_ref.shape, -jnp.inf, jnp.float32)

    for k in range(heads):
        hk = jnp.dot(x_ref[...], w_ref[k], preferred_element_type=jnp.float32)
        cb = (k % 2) * 64
        h_ref[k // 2, :, cb:cb + 64] = hk
        el = jnp.dot(hk, al_ref[k], preferred_element_type=jnp.float32)
        er = jnp.dot(hk, ar_ref[k], preferred_element_type=jnp.float32)
        el_ref[k, 0, 0] = el
        er_ref[k, 0, 0] = er
        gm_ref[0, k] = jnp.maximum(gm_ref[0, k], jnp.max(el))
        gm_ref[1, k] = jnp.maximum(gm_ref[1, k], jnp.max(er))


def _tc_c(unn_ref, es_ref, b1_ref, w2_ref, al2_ref, ar2_ref,
          h2_ref, elr_ref, gm2_ref):
    heads = 2 * unn_ref.shape[0]

    @pl.when(pl.program_id(0) == 0)
    def _():
        gm2_ref[...] = jnp.full(gm2_ref.shape, -jnp.inf, jnp.float32)

    parts = es_ref[...]
    acc = jnp.zeros((unn_ref.shape[1], w2_ref.shape[2]), jnp.float32)
    for k in range(heads):
        c, k4 = k // 4, k % 4
        esum_k = (parts[c * 16 + k4, 0, 0] + parts[c * 16 + 4 + k4, 0, 0]
                  + parts[c * 16 + 8 + k4, 0, 0] + parts[c * 16 + 12 + k4, 0, 0])
        inv_k = 1.0 / (esum_k + 1e-9)
        cb = (k % 2) * 64
        o1 = (unn_ref[k // 2, :, cb:cb + 64] * inv_k[:, None]
              + b1_ref[k][None, :])
        z = jnp.where(o1 > 0, o1, jnp.exp(o1) - 1.0)
        acc = acc + jnp.dot(z, w2_ref[k], preferred_element_type=jnp.float32)
    h2_ref[:, 0:64] = acc
    h2_ref[:, 64:128] = jnp.zeros_like(acc)
    el2 = jnp.dot(acc, al2_ref[0], preferred_element_type=jnp.float32)
    er2 = jnp.dot(acc, ar2_ref[0], preferred_element_type=jnp.float32)
    elr_ref[0, 0, 0] = el2
    elr_ref[1, 0, 0] = er2
    gm2_ref[0] = jnp.maximum(gm2_ref[0], jnp.max(el2))
    gm2_ref[1] = jnp.maximum(gm2_ref[1], jnp.max(er2))


def _tc_e(unn2_ref, es2_ref, b2_ref, out_ref):
    esum = jnp.sum(es2_ref[...], axis=(0, 1, 2))
    inv = 1.0 / (esum + 1e-9)
    u = unn2_ref[0, :, 0:64] + unn2_ref[1, :, 0:64]
    out_ref[...] = u * inv[:, None] + b2_ref[0][None, :]


def _leaky_shift(a, b):
    g = a + b
    return jnp.where(g >= 0.0, g, 0.2 * g)


def _acc_zero(s, zin, acc_sh, sm, lg):
    @pl.when(s < 15)
    def _():
        pltpu.sync_copy(zin.at[pl.ds(0, sm)], acc_sh.at[pl.ds(s * sm, sm)])

    @pl.when(s == 15)
    def _():
        pltpu.sync_copy(zin, acc_sh.at[pl.ds(15 * sm, lg)])


def _acc_out(s, acc_sh, dst_ref, base, sm, lg):
    @pl.when(s < 15)
    def _():
        pltpu.sync_copy(acc_sh.at[pl.ds(s * sm, sm)],
                        dst_ref.at[pl.ds(base + s * sm, sm)])

    @pl.when(s == 15)
    def _():
        pltpu.sync_copy(acc_sh.at[pl.ds(15 * sm, lg)],
                        dst_ref.at[pl.ds(base + 15 * sm, lg)])


def _phase1(n, ept1, base0, head_e_off, gmax, el_v, er_v, esum_v,
            s1_v, d1_v, x1_v, src, dst, exT):
    zero16 = jnp.zeros((16,), jnp.float32)

    def _z(i, _):
        esum_v[pl.ds(i * 16, 16)] = zero16
        return 0
    lax.fori_loop(0, n // 16, _z, 0)

    def _chunk1(i, _):
        b = base0 + i * _C1
        pltpu.sync_copy(src.at[pl.ds(b, _C1)], s1_v)
        pltpu.sync_copy(dst.at[pl.ds(b, _C1)], d1_v)

        def _inner(j, _):
            s16 = s1_v[pl.ds(j * 16, 16)]
            d16 = d1_v[pl.ds(j * 16, 16)]
            ev = plsc.load_gather(el_v, [s16]) + plsc.load_gather(er_v, [d16])
            ev = jnp.where(ev >= 0.0, ev, 0.2 * ev)
            ex = jnp.exp(ev - gmax)
            x1_v[pl.ds(j * 16, 16)] = ex
            plsc.addupdate_scatter(esum_v, [d16], ex)
            return 0
        lax.fori_loop(0, _C1 // 16, _inner, 0)
        pltpu.sync_copy(x1_v, exT.at[pl.ds(head_e_off + b, _C1)])
        return 0
    lax.fori_loop(0, ept1 // _C1, _chunk1, 0)


def _sc_b1(n, e, elT, erT, gm, src, dst, exT, esumP,
           el_v, er_v, esum_v, gm_v, s1_v, d1_v, x1_v):
    c = lax.axis_index("c")
    s = lax.axis_index("s")
    g = c * 16 + s
    ept1 = e // 4
    k_loc = lax.rem(s, 4)
    q = lax.div(s, 4)
    head1 = c * 4 + k_loc
    pltpu.sync_copy(gm.at[pl.ds(head1 * 128, 16)], gm_v.at[pl.ds(0, 16)])
    pltpu.sync_copy(gm.at[pl.ds(1024 + head1 * 128, 16)],
                    gm_v.at[pl.ds(16, 16)])
    pltpu.sync_copy(elT.at[pl.ds(head1 * n, n)], el_v)
    pltpu.sync_copy(erT.at[pl.ds(head1 * n, n)], er_v)
    gmax1 = _leaky_shift(gm_v[pl.ds(0, 16)], gm_v[pl.ds(16, 16)])
    _phase1(n, ept1, q * ept1, head1 * e, gmax1, el_v, er_v, esum_v,
            s1_v, d1_v, x1_v, src, dst, exT)
    pltpu.sync_copy(esum_v, esumP.at[pl.ds(g * n, n)])


def _sc_d1(n, e, elr2, gm2, src, dst, exT2, esumP2,
           el_v, er_v, esum_v, gm_v, s1_v, d1_v, x1_v):
    c = lax.axis_index("c")
    s = lax.axis_index("s")
    g = c * 16 + s
    ept1 = e // 32
    pltpu.sync_copy(gm2.at[pl.ds(0, 16)], gm_v.at[pl.ds(0, 16)])
    pltpu.sync_copy(gm2.at[pl.ds(128, 16)], gm_v.at[pl.ds(16, 16)])
    pltpu.sync_copy(elr2.at[pl.ds(0, n)], el_v)
    pltpu.sync_copy(elr2.at[pl.ds(n, n)], er_v)
    gmax2 = _leaky_shift(gm_v[pl.ds(0, 16)], gm_v[pl.ds(16, 16)])
    _phase1(n, ept1, g * ept1, 0, gmax2, el_v, er_v, esum_v,
            s1_v, d1_v, x1_v, src, dst, exT2)
    pltpu.sync_copy(esum_v, esumP2.at[pl.ds(g * n, n)])


def _p2_chunk(ob, off_a, off_b, idx_off, src, dst, exT, h_tbl, acc_sh,
              s2_v, i2_v, d2_v, xa_v, xb_v, bufs, semE, semsG, semsS):
    des = [
        pltpu.async_copy(src.at[pl.ds(ob, _C2)], s2_v, semE),
        pltpu.async_copy(exT.at[pl.ds(off_a + ob, _C2)], xa_v, semE),
    ]
    if off_b is not None:
        des.append(pltpu.async_copy(exT.at[pl.ds(off_b + ob, _C2)], xb_v,
                                    semE))
    for t in range(_NSUB):
        des.append(pltpu.async_copy(dst.at[pl.ds(ob + t * _SUB, _SUB)],
                                    d2_v.at[t], semE))
    for d in des:
        d.wait()

    def _addi(j, _):
        i2_v[pl.ds(j * 16, 16)] = s2_v[pl.ds(j * 16, 16)] + idx_off
        return 0
    lax.fori_loop(0, _C2 // 16, _addi, 0)

    two = off_b is not None

    def _mrow_t(t, buf):
        def _mrow(r, _):
            ridx = jnp.zeros((16,), jnp.int32) + (t * _SUB + r)
            sca = plsc.load_gather(xa_v, [ridx])
            scb = plsc.load_gather(xb_v, [ridx]) if two else sca
            for jj in range(4):
                sl = pl.ds(jj * 16, 16)
                buf[r, sl] = buf[r, sl] * sca
            for jj in range(4, 8):
                sl = pl.ds(jj * 16, 16)
                buf[r, sl] = buf[r, sl] * scb
            return 0
        lax.fori_loop(0, _SUB, _mrow, 0)

    gd = [None, None]
    sd = [None, None]
    gd[0] = pltpu.async_copy(h_tbl.at[i2_v.at[pl.ds(0, _SUB)]], bufs[0],
                             semsG[0])
    for t in range(_NSUB):
        cur = t % 2
        nxt = 1 - cur
        if t < _NSUB - 1:
            if sd[nxt] is not None:
                sd[nxt].wait()
                sd[nxt] = None
            gd[nxt] = pltpu.async_copy(
                h_tbl.at[i2_v.at[pl.ds((t + 1) * _SUB, _SUB)]], bufs[nxt],
                semsG[nxt])
        gd[cur].wait()
        _mrow_t(t, bufs[cur])
        sd[cur] = pltpu.async_copy(bufs[cur], acc_sh.at[d2_v.at[t]],
                                   semsS[cur], add=True)
    for b in (0, 1):
        if sd[b] is not None:
            sd[b].wait()


def _sc_b2(n, e, src, dst, exT, h1p, zin, unn,
           s2_v, i2_v, d2_v, xa_v, xb_v, bufA, bufB, acc_sh,
           semE, semGA, semGB, semSA, semSB):
    c = lax.axis_index("c")
    s = lax.axis_index("s")
    nout = e // _C2
    nmine = lax.div(nout, 16) + jnp.where(s < lax.rem(nout, 16), 1, 0)
    sm = (n // 16) & ~7
    lg = n - 15 * sm
    bufs = (bufA, bufB)

    for p2 in range(2):
        pair = c * 2 + p2
        _acc_zero(s, zin, acc_sh, sm, lg)
        plsc.subcore_barrier()

        def _outer(i, _):
            ob = (s + i * 16) * _C2
            _p2_chunk(ob, 2 * pair * e, (2 * pair + 1) * e, pair * n,
                      src, dst, exT, h1p, acc_sh, s2_v, i2_v, d2_v,
                      xa_v, xb_v, bufs, semE, (semGA, semGB), (semSA, semSB))
            return 0
        lax.fori_loop(0, nmine, _outer, 0)
        plsc.subcore_barrier()
        _acc_out(s, acc_sh, unn, pair * n, sm, lg)
        plsc.subcore_barrier()


def _sc_d2(n, e, src, dst, exT2, h2p, zin, unn2,
           s2_v, i2_v, d2_v, xa_v, bufA, bufB, acc_sh,
           semE, semGA, semGB, semSA, semSB):
    c = lax.axis_index("c")
    s = lax.axis_index("s")
    half = e // 2
    nout = half // _C2
    nmine = lax.div(nout, 16) + jnp.where(s < lax.rem(nout, 16), 1, 0)
    sm = (n // 16) & ~7
    lg = n - 15 * sm
    bufs = (bufA, bufB)

    _acc_zero(s, zin, acc_sh, sm, lg)
    plsc.subcore_barrier()

    def _outer(i, _):
        ob = c * half + (s + i * 16) * _C2
        _p2_chunk(ob, 0, None, 0, src, dst, exT2, h2p, acc_sh,
                  s2_v, i2_v, d2_v, xa_v, xa_v, bufs, semE,
                  (semGA, semGB), (semSA, semSB))
        return 0
    lax.fori_loop(0, nmine, _outer, 0)
    plsc.subcore_barrier()
    _acc_out(s, acc_sh, unn2, c * n, sm, lg)


def kernel(inputs, edge_index, W1, attn_l1, attn_r1, b1,
           W2, attn_l2, attn_r2, b2):
    n, in_size = inputs.shape
    e = edge_index.shape[1]
    h1h, hid = attn_l1.shape
    out_dim = attn_l2.shape[1]
    src = edge_index[0]
    dst = edge_index[1]
    w1r = W1.reshape(in_size, h1h, hid).transpose(1, 0, 2)
    w2r = W2.reshape(h1h, hid, out_dim)
    b1r = b1.reshape(h1h, hid)
    b2r = b2.reshape(1, out_dim)
    grid = n // _NB
    f32 = jnp.float32
    i32 = jnp.int32

    h1p, elT, erT, gm1 = pl.pallas_call(
        _tc_a,
        grid=(grid,),
        in_specs=[
            pl.BlockSpec((_NB, in_size), lambda i: (i, 0)),
            pl.BlockSpec((h1h, in_size, hid), lambda i: (0, 0, 0)),
            pl.BlockSpec((h1h, hid), lambda i: (0, 0)),
            pl.BlockSpec((h1h, hid), lambda i: (0, 0)),
        ],
        out_specs=[
            pl.BlockSpec((h1h // 2, _NB, 128), lambda i: (0, i, 0)),
            pl.BlockSpec((h1h, 1, 1, _NB), lambda i: (0, i, 0, 0)),
            pl.BlockSpec((h1h, 1, 1, _NB), lambda i: (0, i, 0, 0)),
            pl.BlockSpec((2, h1h, 128), lambda i: (0, 0, 0)),
        ],
        out_shape=[
            jax.ShapeDtypeStruct((h1h // 2, n, 128), f32),
            jax.ShapeDtypeStruct((h1h, grid, 1, _NB), f32),
            jax.ShapeDtypeStruct((h1h, grid, 1, _NB), f32),
            jax.ShapeDtypeStruct((2, h1h, 128), f32),
        ],
    )(inputs, w1r, attn_l1, attn_r1)

    sm0 = (n // 16) & ~7
    lgz = n - 15 * sm0
    zin = jnp.zeros((lgz, 128), f32)
    mesh = plsc.VectorSubcoreMesh(core_axis_name="c", subcore_axis_name="s")
    scp = pltpu.CompilerParams(needs_layout_passes=False)

    exT, esumP = pl.kernel(
        functools.partial(_sc_b1, n, e),
        out_type=[
            jax.ShapeDtypeStruct((h1h * e,), f32),
            jax.ShapeDtypeStruct((32 * n,), f32),
        ],
        mesh=mesh,
        compiler_params=scp,
        scratch_types=[
            pltpu.VMEM((n,), f32),
            pltpu.VMEM((n,), f32),
            pltpu.VMEM((n,), f32),
            pltpu.VMEM((32,), f32),
            pltpu.VMEM((_C1,), i32),
            pltpu.VMEM((_C1,), i32),
            pltpu.VMEM((_C1,), f32),
        ],
    )(elT.reshape(-1), erT.reshape(-1), gm1.reshape(-1), src, dst)

    (unn,) = pl.kernel(
        functools.partial(_sc_b2, n, e),
        out_type=[jax.ShapeDtypeStruct((h1h // 2 * n, 128), f32)],
        mesh=mesh,
        compiler_params=scp,
        scratch_types=[
            pltpu.VMEM((_C2,), i32),
            pltpu.VMEM((_C2,), i32),
            pltpu.VMEM((_NSUB, _SUB), i32),
            pltpu.VMEM((_C2,), f32),
            pltpu.VMEM((_C2,), f32),
            pltpu.VMEM((_SUB, 128), f32),
            pltpu.VMEM((_SUB, 128), f32),
            pltpu.VMEM_SHARED((n, 128), f32),
            pltpu.SemaphoreType.DMA,
            pltpu.SemaphoreType.DMA,
            pltpu.SemaphoreType.DMA,
            pltpu.SemaphoreType.DMA,
            pltpu.SemaphoreType.DMA,
        ],
    )(src, dst, exT, h1p.reshape(h1h // 2 * n, 128), zin)

    h2p, elr2, gm2 = pl.pallas_call(
        _tc_c,
        grid=(grid,),
        in_specs=[
            pl.BlockSpec((h1h // 2, _NB, 128), lambda i: (0, i, 0)),
            pl.BlockSpec((32, 1, 1, _NB), lambda i: (0, i, 0, 0)),
            pl.BlockSpec((h1h, hid), lambda i: (0, 0)),
            pl.BlockSpec((h1h, hid, out_dim), lambda i: (0, 0, 0)),
            pl.BlockSpec((1, out_dim), lambda i: (0, 0)),
            pl.BlockSpec((1, out_dim), lambda i: (0, 0)),
        ],
        out_specs=[
            pl.BlockSpec((_NB, 128), lambda i: (i, 0)),
            pl.BlockSpec((2, 1, 1, _NB), lambda i: (0, i, 0, 0)),
            pl.BlockSpec((2, 128), lambda i: (0, 0)),
        ],
        out_shape=[
            jax.ShapeDtypeStruct((n, 128), f32),
            jax.ShapeDtypeStruct((2, grid, 1, _NB), f32),
            jax.ShapeDtypeStruct((2, 128), f32),
        ],
    )(unn.reshape(h1h // 2, n, 128), esumP.reshape(32, grid, 1, _NB),
      b1r, w2r, attn_l2, attn_r2)

    exT2, esumP2 = pl.kernel(
        functools.partial(_sc_d1, n, e),
        out_type=[
            jax.ShapeDtypeStruct((e,), f32),
            jax.ShapeDtypeStruct((32 * n,), f32),
        ],
        mesh=mesh,
        compiler_params=scp,
        scratch_types=[
            pltpu.VMEM((n,), f32),
            pltpu.VMEM((n,), f32),
            pltpu.VMEM((n,), f32),
            pltpu.VMEM((32,), f32),
            pltpu.VMEM((_C1,), i32),
            pltpu.VMEM((_C1,), i32),
            pltpu.VMEM((_C1,), f32),
        ],
    )(elr2.reshape(-1), gm2.reshape(-1), src, dst)

    (unn2,) = pl.kernel(
        functools.partial(_sc_d2, n, e),
        out_type=[jax.ShapeDtypeStruct((2 * n, 128), f32)],
        mesh=mesh,
        compiler_params=scp,
        scratch_types=[
            pltpu.VMEM((_C2,), i32),
            pltpu.VMEM((_C2,), i32),
            pltpu.VMEM((_NSUB, _SUB), i32),
            pltpu.VMEM((_C2,), f32),
            pltpu.VMEM((_SUB, 128), f32),
            pltpu.VMEM((_SUB, 128), f32),
            pltpu.VMEM_SHARED((n, 128), f32),
            pltpu.SemaphoreType.DMA,
            pltpu.SemaphoreType.DMA,
            pltpu.SemaphoreType.DMA,
            pltpu.SemaphoreType.DMA,
            pltpu.SemaphoreType.DMA,
        ],
    )(src, dst, exT2, h2p, zin)

    out = pl.pallas_call(
        _tc_e,
        grid=(grid,),
        in_specs=[
            pl.BlockSpec((2, _NB, 128), lambda i: (0, i, 0)),
            pl.BlockSpec((32, 1, 1, _NB), lambda i: (0, i, 0, 0)),
            pl.BlockSpec((1, out_dim), lambda i: (0, 0)),
        ],
        out_specs=pl.BlockSpec((_NB, out_dim), lambda i: (i, 0)),
        out_shape=jax.ShapeDtypeStruct((n, out_dim), f32),
    )(unn2.reshape(2, n, 128), esumP2.reshape(32, grid, 1, _NB), b2r)

    return out
